# baseline (speedup 1.0000x reference)
import functools

import jax
import jax.numpy as jnp
from jax import lax
from jax.experimental import pallas as pl
from jax.experimental.pallas import tpu as pltpu
from jax.experimental.pallas import tpu_sc as plsc

B = 4
S = 8192
D = 1024
NC = 2
NS = 16
NW = NC * NS
ROWS_PER_W = S // NW
CR = 16
CHUNK = CR * D
T_CHUNKS = ROWS_PER_W // CR
N_ITERS = T_CHUNKS * B
NBUF = 5
LOOKAHEAD = 2


def _sc_body(x_hbm, tab_hbm, out_hbm, xbuf, tabv, in_s, out_s, tab_s):
    wid = lax.axis_index("s") * NC + lax.axis_index("c")
    base0 = wid * (ROWS_PER_W * D)

    def off(k):
        t, b = divmod(k, B)
        return b * (S * D) + base0 + t * CHUNK

    h_in, h_out, h_tab = {}, {}, {}

    def start_in(k):
        p = k % NBUF
        if k >= NBUF:
            h_out[k - NBUF].wait()
        h_in[k] = pltpu.async_copy(
            x_hbm.at[pl.ds(off(k), CHUNK)], xbuf[p], in_s[p])

    def start_tab(t):
        h_tab[t] = pltpu.async_copy(
            tab_hbm.at[pl.ds(base0 + t * CHUNK, CHUNK)], tabv[t % 2],
            tab_s[t % 2])

    start_tab(0)
    for j in range(LOOKAHEAD):
        start_in(j)

    for k in range(N_ITERS):
        p = k % NBUF
        t, b = divmod(k, B)
        tp = t % 2
        if b == 0:
            h_tab[t].wait()
            if t + 1 < T_CHUNKS:
                start_tab(t + 1)
        h_in[k].wait()

        @plsc.parallel_loop(0, CHUNK, 16, unroll=8)
        def _add(o):
            plsc.addupdate(xbuf[p].at[pl.ds(o, 16)], tabv[tp][pl.ds(o, 16)])

        h_out[k] = pltpu.async_copy(
            xbuf[p], out_hbm.at[pl.ds(off(k), CHUNK)], out_s[p])
        j = k + LOOKAHEAD
        if j < N_ITERS:
            start_in(j)

    for k in range(N_ITERS - NBUF, N_ITERS):
        h_out[k].wait()


def kernel(x, position_table):
    batch, seq_len, d_model = x.shape
    mesh = plsc.VectorSubcoreMesh(core_axis_name="c", subcore_axis_name="s")
    run = functools.partial(
        pl.kernel,
        out_type=jax.ShapeDtypeStruct((batch * seq_len * d_model,), jnp.float32),
        mesh=mesh,
        scratch_types=[
            [pltpu.VMEM((CHUNK,), jnp.float32)] * NBUF,
            [pltpu.VMEM((CHUNK,), jnp.float32)] * 2,
            [pltpu.SemaphoreType.DMA] * NBUF,
            [pltpu.SemaphoreType.DMA] * NBUF,
            [pltpu.SemaphoreType.DMA] * 2,
        ],
    )(_sc_body)
    out = run(x.reshape(-1), position_table[:seq_len].reshape(-1))
    return out.reshape(x.shape)


# speedup vs baseline: 2.9031x; 2.9031x over previous
import functools

import jax
import jax.numpy as jnp
from jax import lax
from jax.experimental import pallas as pl
from jax.experimental.pallas import tpu as pltpu
from jax.experimental.pallas import tpu_sc as plsc

B = 4
S = 8192
D = 1024
NC = 2
NS = 16
NW = NC * NS
ROWS_PER_W = S // NW
CR = 16
CHUNK = CR * D
T_CHUNKS = ROWS_PER_W // CR
N_ITERS = T_CHUNKS * B
NBUF = 5
LOOKAHEAD = 2


def _sc_body(x_hbm, tab_hbm, out_hbm, xbuf, tabv, in_s, out_s, tab_s):
    wid = lax.axis_index("s") * NC + lax.axis_index("c")
    base0 = wid * (ROWS_PER_W * D)

    def off(k):
        t, b = divmod(k, B)
        return b * S + wid * ROWS_PER_W + t * CR

    h_in, h_out, h_tab = {}, {}, {}

    def start_in(k):
        p = k % NBUF
        if k >= NBUF:
            h_out[k - NBUF].wait()
        h_in[k] = pltpu.async_copy(
            x_hbm.at[pl.ds(off(k), CR)], xbuf[p], in_s[p])

    def start_tab(t):
        h_tab[t] = pltpu.async_copy(
            tab_hbm.at[pl.ds(wid * ROWS_PER_W + t * CR, CR)], tabv[t % 2],
            tab_s[t % 2])

    start_tab(0)
    for j in range(LOOKAHEAD):
        start_in(j)

    for k in range(N_ITERS):
        p = k % NBUF
        t, b = divmod(k, B)
        tp = t % 2
        if b == 0:
            h_tab[t].wait()
            if t + 1 < T_CHUNKS:
                start_tab(t + 1)
        h_in[k].wait()

        @plsc.parallel_loop(0, CHUNK, 16, unroll=8)
        def _add(o):
            r = o // D
            c = lax.rem(o, D)
            plsc.addupdate(xbuf[p].at[r, pl.ds(c, 16)], tabv[tp][r, pl.ds(c, 16)])

        h_out[k] = pltpu.async_copy(
            xbuf[p], out_hbm.at[pl.ds(off(k), CR)], out_s[p])
        j = k + LOOKAHEAD
        if j < N_ITERS:
            start_in(j)

    for k in range(N_ITERS - NBUF, N_ITERS):
        h_out[k].wait()


def kernel(x, position_table):
    batch, seq_len, d_model = x.shape
    mesh = plsc.VectorSubcoreMesh(core_axis_name="c", subcore_axis_name="s")
    run = functools.partial(
        pl.kernel,
        out_type=jax.ShapeDtypeStruct((batch * seq_len, d_model), jnp.float32),
        mesh=mesh,
        scratch_types=[
            [pltpu.VMEM((CR, D), jnp.float32)] * NBUF,
            [pltpu.VMEM((CR, D), jnp.float32)] * 2,
            [pltpu.SemaphoreType.DMA] * NBUF,
            [pltpu.SemaphoreType.DMA] * NBUF,
            [pltpu.SemaphoreType.DMA] * 2,
        ],
    )(_sc_body)
    out = run(x.reshape(batch * seq_len, d_model), position_table[:seq_len])
    return out.reshape(x.shape)


# speedup vs baseline: 3.0727x; 1.0584x over previous
import functools

import jax
import jax.numpy as jnp
from jax import lax
from jax.experimental import pallas as pl
from jax.experimental.pallas import tpu as pltpu
from jax.experimental.pallas import tpu_sc as plsc

B = 4
S = 8192
D = 1024
NC = 2
NS = 16
NW = NC * NS
ROWS_PER_W = S // NW
CR = 16
CHUNK = CR * D
T_CHUNKS = ROWS_PER_W // CR
N_ITERS = T_CHUNKS * B
NBUF = 5
LOOKAHEAD = 3


def _sc_body(x_hbm, tab_hbm, out_hbm, xbuf, tabv, in_s, out_s, tab_s):
    wid = lax.axis_index("s") * NC + lax.axis_index("c")

    def off(k):
        t, b = divmod(k, B)
        return b * S + wid * ROWS_PER_W + t * CR

    h_in, h_out, h_tab = {}, {}, {}

    def start_in(k):
        p = k % NBUF
        if k >= NBUF:
            h_out[k - NBUF].wait()
        h_in[k] = pltpu.async_copy(
            x_hbm.at[pl.ds(off(k), CR)], xbuf[p], in_s[p])

    def start_tab(t):
        h_tab[t] = pltpu.async_copy(
            tab_hbm.at[pl.ds(wid * ROWS_PER_W + t * CR, CR)], tabv[t % 2],
            tab_s[t % 2])

    start_tab(0)
    for j in range(LOOKAHEAD):
        start_in(j)

    for k in range(N_ITERS):
        p = k % NBUF
        t, b = divmod(k, B)
        tp = t % 2
        if b == 0:
            h_tab[t].wait()
            if t + 1 < T_CHUNKS:
                start_tab(t + 1)
        h_in[k].wait()
        j = k + LOOKAHEAD
        if j < N_ITERS:
            start_in(j)

        @plsc.parallel_loop(0, CHUNK, 16, unroll=8)
        def _add(o):
            r = o // D
            c = lax.rem(o, D)
            plsc.addupdate(xbuf[p].at[r, pl.ds(c, 16)], tabv[tp][r, pl.ds(c, 16)])

        h_out[k] = pltpu.async_copy(
            xbuf[p], out_hbm.at[pl.ds(off(k), CR)], out_s[p])

    for k in range(N_ITERS - NBUF, N_ITERS):
        h_out[k].wait()


def kernel(x, position_table):
    batch, seq_len, d_model = x.shape
    mesh = plsc.VectorSubcoreMesh(core_axis_name="c", subcore_axis_name="s")
    run = functools.partial(
        pl.kernel,
        out_type=jax.ShapeDtypeStruct((batch * seq_len, d_model), jnp.float32),
        mesh=mesh,
        scratch_types=[
            [pltpu.VMEM((CR, D), jnp.float32)] * NBUF,
            [pltpu.VMEM((CR, D), jnp.float32)] * 2,
            [pltpu.SemaphoreType.DMA] * NBUF,
            [pltpu.SemaphoreType.DMA] * NBUF,
            [pltpu.SemaphoreType.DMA] * 2,
        ],
    )(_sc_body)
    out = run(x.reshape(batch * seq_len, d_model), position_table[:seq_len])
    return out.reshape(x.shape)
